# Initial kernel scaffold; baseline (speedup 1.0000x reference)
import jax
import jax.numpy as jnp
from jax.experimental import pallas as pl


def kernel(features, edges, W1, b1, W2, b2, Wf1, bf1, Wf2, bf2, Wl, bl):
    raise NotImplementedError("write your pallas kernel here")


# speedup vs baseline: 29.9655x; 29.9655x over previous
import functools

import jax
import jax.numpy as jnp
from jax import lax
from jax.experimental import pallas as pl
from jax.experimental.pallas import tpu as pltpu
from jax.experimental.pallas import tpu_sc as plsc

N = 10000
NPAD = 10240
TRASH = NPAD - N
E = 320000
C = 128
TILES = 32
CHUNKS = (E + TILES * C - 1) // (TILES * C)
EPAD = TILES * CHUNKS * C
ROWS_PER_TILE = NPAD // 16

@functools.cache
def _make_deg():
  mesh = plsc.VectorSubcoreMesh(core_axis_name="c", subcore_axis_name="s")

  @functools.partial(
      pl.kernel,
      out_type=jax.ShapeDtypeStruct((2, NPAD, 1), jnp.float32),
      mesh=mesh,
      compiler_params=pltpu.CompilerParams(use_tc_tiling_on_sc=False),
      scratch_types=[
          pltpu.VMEM((CHUNKS, C), jnp.int32),
          pltpu.VMEM((C, 1), jnp.float32),
          pltpu.VMEM((ROWS_PER_TILE, 1), jnp.float32),
          pltpu.VMEM_SHARED((NPAD, 1), jnp.float32),
      ],
  )
  def _deg_kernel(ones_c_hbm, ones_rows_hbm, dst_hbm, out_hbm,
                  dst_v, ones_v, stage_v, acc_sh):
      c = lax.axis_index("c")
      s = lax.axis_index("s")
      t = c * 16 + s
      base = s * ROWS_PER_TILE
      pltpu.sync_copy(dst_hbm.at[t], dst_v)
      pltpu.sync_copy(ones_c_hbm, ones_v)
      pltpu.sync_copy(ones_rows_hbm, stage_v)
      pltpu.sync_copy(stage_v, acc_sh.at[pl.ds(base, ROWS_PER_TILE)])
      plsc.subcore_barrier()

      @pl.loop(0, CHUNKS)
      def _(j):
          pltpu.sync_copy(ones_v, acc_sh.at[dst_v.at[j]], add=True)

      plsc.subcore_barrier()
      pltpu.sync_copy(acc_sh.at[pl.ds(base, ROWS_PER_TILE)], stage_v)
      pltpu.sync_copy(stage_v, out_hbm.at[c, pl.ds(base, ROWS_PER_TILE)])

  return _deg_kernel


@functools.cache
def _make_prop(D):
    mesh = plsc.VectorSubcoreMesh(core_axis_name="c", subcore_axis_name="s")

    @functools.partial(
        pl.kernel,
        out_type=jax.ShapeDtypeStruct((2, NPAD, D), jnp.float32),
        mesh=mesh,
        compiler_params=pltpu.CompilerParams(use_tc_tiling_on_sc=False),
        scratch_types=[
            pltpu.VMEM((CHUNKS, C), jnp.int32),
            pltpu.VMEM((CHUNKS, C), jnp.int32),
            pltpu.VMEM((C, D), jnp.float32),
            pltpu.VMEM((ROWS_PER_TILE, D), jnp.float32),
            pltpu.VMEM_SHARED((NPAD, D), jnp.float32),
            pltpu.SemaphoreType.DMA,
        ],
    )
    def prop(g_hbm, src_hbm, dst_hbm, out_hbm,
             src_v, dst_v, rows_v, stage_v, acc_sh, sem):
        c = lax.axis_index("c")
        s = lax.axis_index("s")
        t = c * 16 + s
        base = s * ROWS_PER_TILE
        pltpu.sync_copy(src_hbm.at[t], src_v)
        pltpu.sync_copy(dst_hbm.at[t], dst_v)
        pltpu.sync_copy(g_hbm.at[pl.ds(base, ROWS_PER_TILE)], stage_v)
        pltpu.sync_copy(stage_v, acc_sh.at[pl.ds(base, ROWS_PER_TILE)])
        plsc.subcore_barrier()

        @pl.loop(0, CHUNKS)
        def _(j):
            pltpu.async_copy(g_hbm.at[src_v.at[j]], rows_v, sem).wait()
            pltpu.sync_copy(rows_v, acc_sh.at[dst_v.at[j]], add=True)

        plsc.subcore_barrier()
        pltpu.sync_copy(acc_sh.at[pl.ds(base, ROWS_PER_TILE)], stage_v)
        pltpu.sync_copy(stage_v, out_hbm.at[c, pl.ds(base, ROWS_PER_TILE)])

    return prop


_B = 1024


def _k2_body(x_ref, a_ref, w_ref, g_ref, dis_ref):
    i = pl.program_id(0)
    deg = a_ref[0] + a_ref[1] - 1.0
    row = lax.broadcasted_iota(jnp.int32, (_B, 1), 0) + i * _B
    dis = jnp.where(row < N, lax.rsqrt(deg), 0.0)
    dis_ref[...] = dis
    g_ref[...] = jnp.dot(x_ref[...], w_ref[...],
                         preferred_element_type=jnp.float32) * dis


def _k2(xp, acc_deg, W1):
    return pl.pallas_call(
        _k2_body,
        grid=(NPAD // _B,),
        in_specs=[
            pl.BlockSpec((_B, 128), lambda i: (i, 0)),
            pl.BlockSpec((2, _B, 1), lambda i: (0, i, 0)),
            pl.BlockSpec((128, 64), lambda i: (0, 0)),
        ],
        out_specs=[
            pl.BlockSpec((_B, 64), lambda i: (i, 0)),
            pl.BlockSpec((_B, 1), lambda i: (i, 0)),
        ],
        out_shape=[
            jax.ShapeDtypeStruct((NPAD, 64), jnp.float32),
            jax.ShapeDtypeStruct((NPAD, 1), jnp.float32),
        ],
    )(xp, acc_deg, W1)


def _k4_body(a_ref, g1_ref, dis_ref, w_ref, b_ref, g2_ref):
    s1 = a_ref[0] + a_ref[1] - g1_ref[...]
    dis = dis_ref[...]
    h1 = jnp.maximum(dis * s1 + b_ref[...], 0.0)
    g2_ref[...] = jnp.dot(h1, w_ref[...],
                          preferred_element_type=jnp.float32) * dis


def _k4(acc1, g1, dis, W2, b1):
    return pl.pallas_call(
        _k4_body,
        grid=(NPAD // _B,),
        in_specs=[
            pl.BlockSpec((2, _B, 64), lambda i: (0, i, 0)),
            pl.BlockSpec((_B, 64), lambda i: (i, 0)),
            pl.BlockSpec((_B, 1), lambda i: (i, 0)),
            pl.BlockSpec((64, 32), lambda i: (0, 0)),
            pl.BlockSpec((1, 64), lambda i: (0, 0)),
        ],
        out_specs=pl.BlockSpec((_B, 32), lambda i: (i, 0)),
        out_shape=jax.ShapeDtypeStruct((NPAD, 32), jnp.float32),
    )(acc1, g1, dis, W2, b1)


def _k6_body(a_ref, g2_ref, dis_ref, b2_ref, wf1_ref, bf1_ref, wf2_ref,
             bf2_ref, wl_ref, bl_ref, emb_ref, pen_ref, pred_ref):
    s2 = a_ref[0] + a_ref[1] - g2_ref[...]
    h2 = dis_ref[...] * s2 + b2_ref[...]
    a1 = jnp.tanh(jnp.dot(h2, wf1_ref[...],
                          preferred_element_type=jnp.float32) + bf1_ref[...])
    lg = jnp.dot(a1, wf2_ref[...],
                 preferred_element_type=jnp.float32) + bf2_ref[...]
    row = lax.broadcasted_iota(jnp.int32, (NPAD, 1), 0)
    lg = jnp.where(row < N, lg, -1e30)
    m = jnp.max(lg, axis=0, keepdims=True)
    e = jnp.exp(lg - m)
    z = jnp.sum(e, axis=0, keepdims=True)
    att = e / z
    emb = lax.dot_general(att, h2, (((0,), (0,)), ((), ())),
                          preferred_element_type=jnp.float32)
    emb_ref[...] = emb
    pp = lax.dot_general(att, att, (((0,), (0,)), ((), ())),
                         preferred_element_type=jnp.float32)
    pp = pp - jnp.eye(8, dtype=jnp.float32)
    pen = jnp.sum(jnp.sqrt(jnp.sum(pp * pp, axis=1)))
    pen_ref[...] = jnp.reshape(pen, (1, 1))
    zt = bl_ref[...]
    for i2 in range(8):
        zt = zt + jnp.dot(emb[i2:i2 + 1, :], wl_ref[i2],
                          preferred_element_type=jnp.float32)
    mz = jnp.max(zt, axis=1, keepdims=True)
    lse = jnp.log(jnp.sum(jnp.exp(zt - mz), axis=1, keepdims=True))
    pred_ref[...] = zt - mz - lse


def _k6(acc2, g2, dis, b2, Wf1, bf1, Wf2, bf2, Wl_r, bl):
    return pl.pallas_call(
        _k6_body,
        out_shape=[
            jax.ShapeDtypeStruct((8, 32), jnp.float32),
            jax.ShapeDtypeStruct((1, 1), jnp.float32),
            jax.ShapeDtypeStruct((1, 10), jnp.float32),
        ],
    )(acc2, g2, dis, b2, Wf1, bf1, Wf2, bf2, Wl_r, bl)


def kernel(features, edges, W1, b1, W2, b2, Wf1, bf1, Wf2, bf2, Wl, bl):
    f32 = jnp.float32
    xp = jnp.pad(features, ((0, NPAD - N), (0, 0)))
    npad = EPAD - E
    trash = N + (jnp.arange(npad, dtype=jnp.int32) % TRASH)
    src = jnp.concatenate([edges[0], trash]).reshape(TILES, CHUNKS, C)
    dst = jnp.concatenate([edges[1], trash]).reshape(TILES, CHUNKS, C)
    ones_c = jnp.ones((C, 1), f32)
    ones_rows = jnp.ones((ROWS_PER_TILE, 1), f32)

    acc_deg = _make_deg()(ones_c, ones_rows, dst)
    g1, dis = _k2(xp, acc_deg, W1)
    acc1 = _make_prop(64)(g1, src, dst)
    g2 = _k4(acc1, g1, dis, W2, b1.reshape(1, 64))
    acc2 = _make_prop(32)(g2, src, dst)
    emb, pen, pred = _k6(acc2, g2, dis, b2.reshape(1, 32), Wf1,
                         bf1.reshape(1, 16), Wf2, bf2.reshape(1, 8),
                         Wl.reshape(8, 32, 10), bl.reshape(1, 10))
    return (emb.reshape(1, 256), pen[0, 0], pred)
